# baseline (speedup 1.0000x reference)
import functools

import jax
import jax.numpy as jnp
from jax import lax
from jax.experimental import pallas as pl
from jax.experimental.pallas import tpu as pltpu
from jax.experimental.pallas import tpu_sc as plsc

N = 10000
E = 320000
D_IN = 128
D_HID = 64
D_HALF = D_HID // 2

NC = 2
NS = 16
KA = 128
NB = -(-E // (NS * KA))
E_PAD = NS * KA * NB
N_ACC = 10048
RPS = N_ACC // NS
NCHUNK = 4
CH = RPS // NCHUNK
L = 16


def _sc_mesh():
    return plsc.VectorSubcoreMesh(core_axis_name="c", subcore_axis_name="s",
                                  num_cores=NC, num_subcores=NS)


def _rsqrt16(x):
    i = plsc.bitcast(x, jnp.int32)
    i = jnp.int32(0x5F3759DF) - (i >> 1)
    y = plsc.bitcast(i, jnp.float32)
    y = y * (1.5 - 0.5 * x * y * y)
    y = y * (1.5 - 0.5 * x * y * y)
    return y


@functools.lru_cache(maxsize=None)
def _make_agg(scale_stage, epi):
    NBUF = 6
    PF = 3

    @functools.partial(
        pl.kernel,
        out_type=jax.ShapeDtypeStruct((NC, NS, NCHUNK, CH, D_HALF),
                                      jnp.float32),
        mesh=_sc_mesh(),
        compiler_params=pltpu.CompilerParams(use_tc_tiling_on_sc=False,
                                             needs_layout_passes=False),
        scratch_types=(
            [pltpu.VMEM((NB, KA), jnp.int32)] * 2
            + [pltpu.VMEM((KA, D_HALF), jnp.float32)] * NBUF
            + [pltpu.VMEM((RPS, 16), jnp.float32)]
            + [pltpu.VMEM((CH, D_HALF), jnp.float32)] * 2
            + [pltpu.VMEM((D_HALF,), jnp.float32)]
            + [pltpu.VMEM_SHARED((N_ACC, D_HALF), jnp.float32)] * 2
            + [pltpu.SemaphoreType.DMA] * (2 * NBUF)
        ),
    )
    def agg(p_hbm, src_hbm, dst_hbm, dinv_hbm, b_hbm, out_hbm,
            src_v, dst_v, *rest):
        rows = rest[0:NBUF]
        dv_v = rest[NBUF]
        cb0, cb1 = rest[NBUF + 1], rest[NBUF + 2]
        b_v = rest[NBUF + 3]
        acc_sh = rest[NBUF + 4]
        tab = rest[NBUF + 5]
        gsem = rest[NBUF + 6:2 * NBUF + 6]
        ssem = rest[2 * NBUF + 6:3 * NBUF + 6]
        cid = lax.axis_index("c")
        sid = lax.axis_index("s")
        base = sid * RPS
        pltpu.sync_copy(src_hbm.at[sid], src_v)
        pltpu.sync_copy(dst_hbm.at[sid], dst_v)
        pltpu.sync_copy(dinv_hbm.at[sid], dv_v)
        pltpu.sync_copy(b_hbm.at[cid], b_v)

        zero16 = jnp.zeros((L,), jnp.float32)

        def zbody(r, carry):
            cb1[r, pl.ds(0, L)] = zero16
            cb1[r, pl.ds(L, L)] = zero16
            return carry

        lax.fori_loop(0, CH, zbody, 0)
        for ck in range(NCHUNK):
            pltpu.sync_copy(cb1, acc_sh.at[pl.ds(base + ck * CH, CH)])

        for ck in range(NCHUNK):
            off = base + ck * CH
            if scale_stage:
                pltpu.sync_copy(p_hbm.at[cid, sid, ck], cb0)

                def sbody(r, carry, _ck=ck):
                    dv = dv_v[_ck * CH + r]
                    cb0[r, pl.ds(0, L)] = cb0[r, pl.ds(0, L)] * dv
                    cb0[r, pl.ds(L, L)] = cb0[r, pl.ds(L, L)] * dv
                    return carry

                lax.fori_loop(0, CH, sbody, 0)
                pltpu.sync_copy(cb0, tab.at[pl.ds(off, CH)])
            else:
                pltpu.sync_copy(p_hbm.at[cid, sid, ck], tab.at[pl.ds(off, CH)])
        plsc.subcore_barrier()

        def gather(j, b):
            pltpu.async_copy(tab.at[src_v.at[j]], rows[b], gsem[b])

        def gwait(j, b):
            pltpu.make_async_copy(tab.at[src_v.at[j]], rows[b], gsem[b]).wait()

        def scat(j, b):
            pltpu.async_copy(rows[b], acc_sh.at[dst_v.at[j]], ssem[b], add=True)

        def swait(j, b):
            pltpu.make_async_copy(rows[b], acc_sh.at[dst_v.at[j]],
                                  ssem[b]).wait()

        for j in range(PF):
            gather(j, j % NBUF)
        for j in range(NBUF):
            if j + PF < NB:
                bp = (j + PF) % NBUF
                if j + PF >= NBUF:
                    swait(j + PF - NBUF, bp)
                gather(j + PF, bp)
            gwait(j, j % NBUF)
            scat(j, j % NBUF)

        def body(t, carry):
            for b in range(NBUF):
                j = NBUF * t + b
                bp = (b + PF) % NBUF
                swait(j + PF - NBUF, bp)
                gather(j + PF, bp)
                gwait(j, b)
                scat(j, b)
            return carry

        n_main = (NB - PF) // NBUF
        lax.fori_loop(1, n_main, body, 0)
        for j in range(NBUF * n_main, NB):
            b = j % NBUF
            if j + PF < NB:
                bp = (j + PF) % NBUF
                swait(j + PF - NBUF, bp)
                gather(j + PF, bp)
            gwait(j, b)
            scat(j, b)
        for j in range(NB - NBUF, NB):
            swait(j, j % NBUF)
        plsc.subcore_barrier()

        for ck in range(NCHUNK):
            off = base + ck * CH
            pltpu.sync_copy(acc_sh.at[pl.ds(off, CH)], cb0)
            pltpu.sync_copy(tab.at[pl.ds(off, CH)], cb1)

            def ebody(r, carry, _ck=ck):
                dv = dv_v[_ck * CH + r]
                for half in range(2):
                    sl = pl.ds(half * L, L)
                    u = cb0[r, sl] + cb1[r, sl]
                    if epi == "p2":
                        v = (u * dv + b_v[sl]) * dv
                    else:
                        v = u * dv
                    cb0[r, sl] = v
                return carry

            lax.fori_loop(0, CH, ebody, 0)
            pltpu.sync_copy(cb0, out_hbm.at[cid, sid, ck])

    return agg


@functools.lru_cache(maxsize=None)
def _make_dinv():
    @functools.partial(
        pl.kernel,
        out_type=jax.ShapeDtypeStruct((NS, RPS, 16), jnp.float32),
        mesh=_sc_mesh(),
        compiler_params=pltpu.CompilerParams(use_tc_tiling_on_sc=False,
                                             needs_layout_passes=False),
        scratch_types=[
            pltpu.VMEM((NB, KA), jnp.int32),
            pltpu.VMEM((KA, 16), jnp.float32),
            pltpu.VMEM((RPS, 16), jnp.float32),
            pltpu.VMEM_SHARED((N_ACC, 16), jnp.float32),
        ],
    )
    def dinv(dst_hbm, ones_hbm, out_hbm, dst_v, ones_v, db, acc_sh):
        cid = lax.axis_index("c")
        sid = lax.axis_index("s")
        pltpu.sync_copy(dst_hbm.at[sid], dst_v)
        pltpu.sync_copy(ones_hbm, ones_v)

        zero16 = jnp.zeros((16,), jnp.float32)

        def zbody(r, carry):
            db[r] = zero16
            return carry

        lax.fori_loop(0, RPS, zbody, 0)
        pltpu.sync_copy(db, acc_sh.at[pl.ds(sid * RPS, RPS)])
        plsc.subcore_barrier()

        def body(j, carry):
            pltpu.sync_copy(ones_v, acc_sh.at[dst_v.at[j]], add=True)
            return carry

        lax.fori_loop(0, NB, body, 0)
        plsc.subcore_barrier()
        pltpu.sync_copy(acc_sh.at[pl.ds(sid * RPS, RPS)], db)

        def rbody(r, carry):
            db[r] = _rsqrt16(db[r] + 1.0)
            return carry

        lax.fori_loop(0, RPS, rbody, 0)

        @pl.when(cid == 0)
        def _():
            pltpu.sync_copy(db, out_hbm.at[sid])

    return dinv


_R = 2000


def _proj_body(x_ref, w1_ref, h0_ref):
    v = lax.dot_general(x_ref[...], w1_ref[...], (((1,), (1,)), ((), ())),
                        preferred_element_type=jnp.float32)
    h0_ref[0] = v[:, :D_HALF]
    h0_ref[1] = v[:, D_HALF:]


def _out_body(g0_ref, g1_ref, w2_ref, b2_ref, w3_ref, b3_ref, m_ref, s_ref):
    g = jnp.concatenate([g0_ref[0], g1_ref[0]], axis=-1)
    dims = (((1,), (1,)), ((), ()))
    m_ref[...] = lax.dot_general(g, w2_ref[...], dims,
                                 preferred_element_type=jnp.float32) + b2_ref[...]
    s_ref[...] = lax.dot_general(g, w3_ref[...], dims,
                                 preferred_element_type=jnp.float32) + b3_ref[...]


def _row_spec(d):
    return pl.BlockSpec((_R, d), lambda i: (i, 0))


def _part_spec(c, d):
    return pl.BlockSpec((1, _R, d), lambda i, _c=c: (_c, i, 0))


def _full_spec(shape):
    return pl.BlockSpec(shape, lambda i: (0,) * len(shape))


def _proj(x, W1):
    return pl.pallas_call(
        _proj_body,
        grid=(N // _R,),
        in_specs=[_row_spec(D_IN), _full_spec(W1.shape)],
        out_specs=pl.BlockSpec((NC, _R, D_HALF), lambda i: (0, i, 0)),
        out_shape=jax.ShapeDtypeStruct((NC, N_ACC, D_HALF), jnp.float32),
    )(x, W1)


def _out_stage(g, W2, b2, W3, b3):
    return pl.pallas_call(
        _out_body,
        grid=(N // _R,),
        in_specs=[_part_spec(0, D_HALF), _part_spec(1, D_HALF),
                  _full_spec(W2.shape), _full_spec(b2.shape),
                  _full_spec(W3.shape), _full_spec(b3.shape)],
        out_specs=[_row_spec(D_IN), _row_spec(D_IN)],
        out_shape=[jax.ShapeDtypeStruct((N, D_IN), jnp.float32),
                   jax.ShapeDtypeStruct((N, D_IN), jnp.float32)],
    )(g, g, W2, b2, W3, b3)


def kernel(x, edge_index, W1, b1, W2, b2, W3, b3):
    src = edge_index[0].astype(jnp.int32)
    dst = edge_index[1].astype(jnp.int32)
    src_agg = jnp.concatenate(
        [src, jnp.zeros((E_PAD - E,), jnp.int32)]).reshape(NS, NB, KA)
    dst_agg = jnp.concatenate(
        [dst, jnp.full((E_PAD - E,), N, jnp.int32)]).reshape(NS, NB, KA)

    ones16 = jnp.ones((KA, 16), jnp.float32)
    b1c = b1.reshape(NC, D_HALF)
    shape5 = (NC, NS, NCHUNK, CH, D_HALF)

    dinv = _make_dinv()(dst_agg, ones16)
    h0 = _proj(x, W1)
    h0r = h0.reshape(shape5)

    p2 = _make_agg(True, "p2")(h0r, src_agg, dst_agg, dinv, b1c)
    g = _make_agg(False, "g")(p2, src_agg, dst_agg, dinv, b1c)

    m, s = _out_stage(g.reshape(NC, N_ACC, D_HALF),
                      W2, b2.reshape(1, D_IN), W3, b3.reshape(1, D_IN))
    return (m, s)


# speedup vs baseline: 1.0045x; 1.0045x over previous
import functools

import jax
import jax.numpy as jnp
from jax import lax
from jax.experimental import pallas as pl
from jax.experimental.pallas import tpu as pltpu
from jax.experimental.pallas import tpu_sc as plsc

N = 10000
E = 320000
D_IN = 128
D_HID = 64
D_HALF = D_HID // 2

NC = 2
NS = 16
KA = 128
NB = -(-E // (NS * KA))
E_PAD = NS * KA * NB
N_ACC = 10048
RPS = N_ACC // NS
NCHUNK = 4
CH = RPS // NCHUNK
L = 16


def _sc_mesh():
    return plsc.VectorSubcoreMesh(core_axis_name="c", subcore_axis_name="s",
                                  num_cores=NC, num_subcores=NS)


def _rsqrt16(x):
    i = plsc.bitcast(x, jnp.int32)
    i = jnp.int32(0x5F3759DF) - (i >> 1)
    y = plsc.bitcast(i, jnp.float32)
    y = y * (1.5 - 0.5 * x * y * y)
    y = y * (1.5 - 0.5 * x * y * y)
    return y


@functools.lru_cache(maxsize=None)
def _make_agg(scale_stage, epi):
    NBUF = 4
    PF = 2

    @functools.partial(
        pl.kernel,
        out_type=jax.ShapeDtypeStruct((NC, NS, NCHUNK, CH, D_HALF),
                                      jnp.float32),
        mesh=_sc_mesh(),
        compiler_params=pltpu.CompilerParams(use_tc_tiling_on_sc=False,
                                             needs_layout_passes=False),
        scratch_types=(
            [pltpu.VMEM((NB, KA), jnp.int32)] * 2
            + [pltpu.VMEM((KA, D_HALF), jnp.float32)] * NBUF
            + [pltpu.VMEM((RPS, 16), jnp.float32)]
            + [pltpu.VMEM((CH, D_HALF), jnp.float32)] * 2
            + [pltpu.VMEM((D_HALF,), jnp.float32)]
            + [pltpu.VMEM_SHARED((N_ACC, D_HALF), jnp.float32)] * 2
            + [pltpu.SemaphoreType.DMA] * (2 * NBUF)
        ),
    )
    def agg(p_hbm, src_hbm, dst_hbm, dinv_hbm, b_hbm, out_hbm,
            src_v, dst_v, *rest):
        rows = rest[0:NBUF]
        dv_v = rest[NBUF]
        cb0, cb1 = rest[NBUF + 1], rest[NBUF + 2]
        b_v = rest[NBUF + 3]
        acc_sh = rest[NBUF + 4]
        tab = rest[NBUF + 5]
        gsem = rest[NBUF + 6:2 * NBUF + 6]
        ssem = rest[2 * NBUF + 6:3 * NBUF + 6]
        cid = lax.axis_index("c")
        sid = lax.axis_index("s")
        base = sid * RPS
        pltpu.sync_copy(src_hbm.at[sid], src_v)
        pltpu.sync_copy(dst_hbm.at[sid], dst_v)
        pltpu.sync_copy(dinv_hbm.at[sid], dv_v)
        pltpu.sync_copy(b_hbm.at[cid], b_v)

        zero16 = jnp.zeros((L,), jnp.float32)

        def zbody(r, carry):
            cb1[r, pl.ds(0, L)] = zero16
            cb1[r, pl.ds(L, L)] = zero16
            return carry

        lax.fori_loop(0, CH, zbody, 0)
        for ck in range(NCHUNK):
            pltpu.sync_copy(cb1, acc_sh.at[pl.ds(base + ck * CH, CH)])

        for ck in range(NCHUNK):
            off = base + ck * CH
            if scale_stage:
                pltpu.sync_copy(p_hbm.at[cid, sid, ck], cb0)

                def sbody(r, carry, _ck=ck):
                    dv = dv_v[_ck * CH + r]
                    cb0[r, pl.ds(0, L)] = cb0[r, pl.ds(0, L)] * dv
                    cb0[r, pl.ds(L, L)] = cb0[r, pl.ds(L, L)] * dv
                    return carry

                lax.fori_loop(0, CH, sbody, 0)
                pltpu.sync_copy(cb0, tab.at[pl.ds(off, CH)])
            else:
                pltpu.sync_copy(p_hbm.at[cid, sid, ck], tab.at[pl.ds(off, CH)])
        plsc.subcore_barrier()

        def gather(j, b):
            pltpu.async_copy(tab.at[src_v.at[j]], rows[b], gsem[b])

        def gwait(j, b):
            pltpu.make_async_copy(tab.at[src_v.at[j]], rows[b], gsem[b]).wait()

        def scat(j, b):
            pltpu.async_copy(rows[b], acc_sh.at[dst_v.at[j]], ssem[b], add=True)

        def swait(j, b):
            pltpu.make_async_copy(rows[b], acc_sh.at[dst_v.at[j]],
                                  ssem[b]).wait()

        for j in range(PF):
            gather(j, j % NBUF)
        for j in range(NBUF):
            if j + PF < NB:
                bp = (j + PF) % NBUF
                if j + PF >= NBUF:
                    swait(j + PF - NBUF, bp)
                gather(j + PF, bp)
            gwait(j, j % NBUF)
            scat(j, j % NBUF)

        def body(t, carry):
            for b in range(NBUF):
                j = NBUF * t + b
                bp = (b + PF) % NBUF
                swait(j + PF - NBUF, bp)
                gather(j + PF, bp)
                gwait(j, b)
                scat(j, b)
            return carry

        n_main = (NB - PF) // NBUF
        lax.fori_loop(1, n_main, body, 0)
        for j in range(NBUF * n_main, NB):
            b = j % NBUF
            if j + PF < NB:
                bp = (j + PF) % NBUF
                swait(j + PF - NBUF, bp)
                gather(j + PF, bp)
            gwait(j, b)
            scat(j, b)
        for j in range(NB - NBUF, NB):
            swait(j, j % NBUF)
        plsc.subcore_barrier()

        for ck in range(NCHUNK):
            off = base + ck * CH
            pltpu.sync_copy(acc_sh.at[pl.ds(off, CH)], cb0)
            pltpu.sync_copy(tab.at[pl.ds(off, CH)], cb1)

            def ebody(r, carry, _ck=ck):
                dv = dv_v[_ck * CH + r]
                for half in range(2):
                    sl = pl.ds(half * L, L)
                    u = cb0[r, sl] + cb1[r, sl]
                    if epi == "p2":
                        v = (u * dv + b_v[sl]) * dv
                    else:
                        v = u * dv
                    cb0[r, sl] = v
                return carry

            lax.fori_loop(0, CH, ebody, 0)
            pltpu.sync_copy(cb0, out_hbm.at[cid, sid, ck])

    return agg


@functools.lru_cache(maxsize=None)
def _make_dinv():
    @functools.partial(
        pl.kernel,
        out_type=jax.ShapeDtypeStruct((NS, RPS, 16), jnp.float32),
        mesh=_sc_mesh(),
        compiler_params=pltpu.CompilerParams(use_tc_tiling_on_sc=False,
                                             needs_layout_passes=False),
        scratch_types=[
            pltpu.VMEM((NB, KA), jnp.int32),
            pltpu.VMEM((KA, 16), jnp.float32),
            pltpu.VMEM((RPS, 16), jnp.float32),
            pltpu.VMEM_SHARED((N_ACC, 16), jnp.float32),
        ],
    )
    def dinv(dst_hbm, ones_hbm, out_hbm, dst_v, ones_v, db, acc_sh):
        cid = lax.axis_index("c")
        sid = lax.axis_index("s")
        pltpu.sync_copy(dst_hbm.at[sid], dst_v)
        pltpu.sync_copy(ones_hbm, ones_v)

        zero16 = jnp.zeros((16,), jnp.float32)

        def zbody(r, carry):
            db[r] = zero16
            return carry

        lax.fori_loop(0, RPS, zbody, 0)
        pltpu.sync_copy(db, acc_sh.at[pl.ds(sid * RPS, RPS)])
        plsc.subcore_barrier()

        def body(j, carry):
            pltpu.sync_copy(ones_v, acc_sh.at[dst_v.at[j]], add=True)
            return carry

        lax.fori_loop(0, NB, body, 0)
        plsc.subcore_barrier()
        pltpu.sync_copy(acc_sh.at[pl.ds(sid * RPS, RPS)], db)

        def rbody(r, carry):
            db[r] = _rsqrt16(db[r] + 1.0)
            return carry

        lax.fori_loop(0, RPS, rbody, 0)

        @pl.when(cid == 0)
        def _():
            pltpu.sync_copy(db, out_hbm.at[sid])

    return dinv


_R = 2000


def _proj_body(x_ref, w1_ref, h0_ref):
    v = lax.dot_general(x_ref[...], w1_ref[...], (((1,), (1,)), ((), ())),
                        preferred_element_type=jnp.float32)
    h0_ref[0] = v[:, :D_HALF]
    h0_ref[1] = v[:, D_HALF:]


def _out_body(g0_ref, g1_ref, w2_ref, b2_ref, w3_ref, b3_ref, m_ref, s_ref):
    g = jnp.concatenate([g0_ref[0], g1_ref[0]], axis=-1)
    dims = (((1,), (1,)), ((), ()))
    m_ref[...] = lax.dot_general(g, w2_ref[...], dims,
                                 preferred_element_type=jnp.float32) + b2_ref[...]
    s_ref[...] = lax.dot_general(g, w3_ref[...], dims,
                                 preferred_element_type=jnp.float32) + b3_ref[...]


def _row_spec(d):
    return pl.BlockSpec((_R, d), lambda i: (i, 0))


def _part_spec(c, d):
    return pl.BlockSpec((1, _R, d), lambda i, _c=c: (_c, i, 0))


def _full_spec(shape):
    return pl.BlockSpec(shape, lambda i: (0,) * len(shape))


def _proj(x, W1):
    return pl.pallas_call(
        _proj_body,
        grid=(N // _R,),
        in_specs=[_row_spec(D_IN), _full_spec(W1.shape)],
        out_specs=pl.BlockSpec((NC, _R, D_HALF), lambda i: (0, i, 0)),
        out_shape=jax.ShapeDtypeStruct((NC, N_ACC, D_HALF), jnp.float32),
    )(x, W1)


def _out_stage(g, W2, b2, W3, b3):
    return pl.pallas_call(
        _out_body,
        grid=(N // _R,),
        in_specs=[_part_spec(0, D_HALF), _part_spec(1, D_HALF),
                  _full_spec(W2.shape), _full_spec(b2.shape),
                  _full_spec(W3.shape), _full_spec(b3.shape)],
        out_specs=[_row_spec(D_IN), _row_spec(D_IN)],
        out_shape=[jax.ShapeDtypeStruct((N, D_IN), jnp.float32),
                   jax.ShapeDtypeStruct((N, D_IN), jnp.float32)],
    )(g, g, W2, b2, W3, b3)


def kernel(x, edge_index, W1, b1, W2, b2, W3, b3):
    src = edge_index[0].astype(jnp.int32)
    dst = edge_index[1].astype(jnp.int32)
    src_agg = jnp.concatenate(
        [src, jnp.zeros((E_PAD - E,), jnp.int32)]).reshape(NS, NB, KA)
    dst_agg = jnp.concatenate(
        [dst, jnp.full((E_PAD - E,), N, jnp.int32)]).reshape(NS, NB, KA)

    ones16 = jnp.ones((KA, 16), jnp.float32)
    b1c = b1.reshape(NC, D_HALF)
    shape5 = (NC, NS, NCHUNK, CH, D_HALF)

    dinv = _make_dinv()(dst_agg, ones16)
    h0 = _proj(x, W1)
    h0r = h0.reshape(shape5)

    p2 = _make_agg(True, "p2")(h0r, src_agg, dst_agg, dinv, b1c)
    g = _make_agg(False, "g")(p2, src_agg, dst_agg, dinv, b1c)

    m, s = _out_stage(g.reshape(NC, N_ACC, D_HALF),
                      W2, b2.reshape(1, D_IN), W3, b3.reshape(1, D_IN))
    return (m, s)


# speedup vs baseline: 1.0202x; 1.0156x over previous
import functools

import jax
import jax.numpy as jnp
from jax import lax
from jax.experimental import pallas as pl
from jax.experimental.pallas import tpu as pltpu
from jax.experimental.pallas import tpu_sc as plsc

N = 10000
E = 320000
D_IN = 128
D_HID = 64
D_HALF = D_HID // 2

NC = 2
NS = 16
KA = 160
NB = E // (NS * KA)
N_ACC = 10048
RPS = N_ACC // NS
NCHUNK = 4
CH = RPS // NCHUNK
L = 16


def _sc_mesh():
    return plsc.VectorSubcoreMesh(core_axis_name="c", subcore_axis_name="s",
                                  num_cores=NC, num_subcores=NS)


def _rsqrt16(x):
    i = plsc.bitcast(x, jnp.int32)
    i = jnp.int32(0x5F3759DF) - (i >> 1)
    y = plsc.bitcast(i, jnp.float32)
    y = y * (1.5 - 0.5 * x * y * y)
    y = y * (1.5 - 0.5 * x * y * y)
    return y


@functools.lru_cache(maxsize=None)
def _make_agg(scale_stage, epi):
    NBUF = 4
    PF = 2

    @functools.partial(
        pl.kernel,
        out_type=jax.ShapeDtypeStruct((NC, NS, NCHUNK, CH, D_HALF),
                                      jnp.float32),
        mesh=_sc_mesh(),
        compiler_params=pltpu.CompilerParams(use_tc_tiling_on_sc=False,
                                             needs_layout_passes=False),
        scratch_types=(
            [pltpu.VMEM((NB, KA), jnp.int32)] * 2
            + [pltpu.VMEM((KA, D_HALF), jnp.float32)] * NBUF
            + [pltpu.VMEM((RPS, 16), jnp.float32)]
            + [pltpu.VMEM((CH, D_HALF), jnp.float32)] * 2
            + [pltpu.VMEM((D_HALF,), jnp.float32)]
            + [pltpu.VMEM_SHARED((N_ACC, D_HALF), jnp.float32)] * 2
            + [pltpu.SemaphoreType.DMA] * (2 * NBUF)
        ),
    )
    def agg(p_hbm, src_hbm, dst_hbm, dinv_hbm, b_hbm, out_hbm,
            src_v, dst_v, *rest):
        rows = rest[0:NBUF]
        dv_v = rest[NBUF]
        cb0, cb1 = rest[NBUF + 1], rest[NBUF + 2]
        b_v = rest[NBUF + 3]
        acc_sh = rest[NBUF + 4]
        tab = rest[NBUF + 5]
        gsem = rest[NBUF + 6:2 * NBUF + 6]
        ssem = rest[2 * NBUF + 6:3 * NBUF + 6]
        cid = lax.axis_index("c")
        sid = lax.axis_index("s")
        base = sid * RPS
        pltpu.sync_copy(src_hbm.at[sid], src_v)
        pltpu.sync_copy(dst_hbm.at[sid], dst_v)
        pltpu.sync_copy(dinv_hbm.at[sid], dv_v)
        pltpu.sync_copy(b_hbm.at[cid], b_v)

        zero16 = jnp.zeros((L,), jnp.float32)

        def zbody(r, carry):
            cb1[r, pl.ds(0, L)] = zero16
            cb1[r, pl.ds(L, L)] = zero16
            return carry

        lax.fori_loop(0, CH, zbody, 0)
        for ck in range(NCHUNK):
            pltpu.sync_copy(cb1, acc_sh.at[pl.ds(base + ck * CH, CH)])

        for ck in range(NCHUNK):
            off = base + ck * CH
            if scale_stage:
                pltpu.sync_copy(p_hbm.at[cid, sid, ck], cb0)

                def sbody(r, carry, _ck=ck):
                    dv = dv_v[_ck * CH + r]
                    cb0[r, pl.ds(0, L)] = cb0[r, pl.ds(0, L)] * dv
                    cb0[r, pl.ds(L, L)] = cb0[r, pl.ds(L, L)] * dv
                    return carry

                lax.fori_loop(0, CH, sbody, 0)
                pltpu.sync_copy(cb0, tab.at[pl.ds(off, CH)])
            else:
                pltpu.sync_copy(p_hbm.at[cid, sid, ck], tab.at[pl.ds(off, CH)])
        plsc.subcore_barrier()

        def gather(j, b):
            pltpu.async_copy(tab.at[src_v.at[j]], rows[b], gsem[b])

        def gwait(j, b):
            pltpu.make_async_copy(tab.at[src_v.at[j]], rows[b], gsem[b]).wait()

        def scat(j, b):
            pltpu.async_copy(rows[b], acc_sh.at[dst_v.at[j]], ssem[b], add=True)

        def swait(j, b):
            pltpu.make_async_copy(rows[b], acc_sh.at[dst_v.at[j]],
                                  ssem[b]).wait()

        for j in range(PF):
            gather(j, j % NBUF)
        for j in range(NBUF):
            if j + PF < NB:
                bp = (j + PF) % NBUF
                if j + PF >= NBUF:
                    swait(j + PF - NBUF, bp)
                gather(j + PF, bp)
            gwait(j, j % NBUF)
            scat(j, j % NBUF)

        def body(t, carry):
            for b in range(NBUF):
                j = NBUF * t + b
                bp = (b + PF) % NBUF
                swait(j + PF - NBUF, bp)
                gather(j + PF, bp)
                gwait(j, b)
                scat(j, b)
            return carry

        n_main = (NB - PF) // NBUF
        lax.fori_loop(1, n_main, body, 0)
        for j in range(NBUF * n_main, NB):
            b = j % NBUF
            if j + PF < NB:
                bp = (j + PF) % NBUF
                swait(j + PF - NBUF, bp)
                gather(j + PF, bp)
            gwait(j, b)
            scat(j, b)
        for j in range(NB - NBUF, NB):
            swait(j, j % NBUF)
        plsc.subcore_barrier()

        for ck in range(NCHUNK):
            off = base + ck * CH
            pltpu.sync_copy(acc_sh.at[pl.ds(off, CH)], cb0)
            pltpu.sync_copy(tab.at[pl.ds(off, CH)], cb1)

            def ebody(r, carry, _ck=ck):
                dv = dv_v[_ck * CH + r]
                for half in range(2):
                    sl = pl.ds(half * L, L)
                    u = cb0[r, sl] + cb1[r, sl]
                    if epi == "p2":
                        v = (u * dv + b_v[sl]) * dv
                    else:
                        v = u * dv
                    cb0[r, sl] = v
                return carry

            lax.fori_loop(0, CH, ebody, 0)
            pltpu.sync_copy(cb0, out_hbm.at[cid, sid, ck])

    return agg


@functools.lru_cache(maxsize=None)
def _make_dinv():
    @functools.partial(
        pl.kernel,
        out_type=jax.ShapeDtypeStruct((NS, RPS, 16), jnp.float32),
        mesh=_sc_mesh(),
        compiler_params=pltpu.CompilerParams(use_tc_tiling_on_sc=False,
                                             needs_layout_passes=False),
        scratch_types=[
            pltpu.VMEM((NB, KA), jnp.int32),
            pltpu.VMEM((KA, 16), jnp.float32),
            pltpu.VMEM((RPS, 16), jnp.float32),
            pltpu.VMEM_SHARED((N_ACC, 16), jnp.float32),
        ],
    )
    def dinv(dst_hbm, ones_hbm, out_hbm, dst_v, ones_v, db, acc_sh):
        cid = lax.axis_index("c")
        sid = lax.axis_index("s")
        pltpu.sync_copy(dst_hbm.at[sid], dst_v)
        pltpu.sync_copy(ones_hbm, ones_v)

        zero16 = jnp.zeros((16,), jnp.float32)

        def zbody(r, carry):
            db[r] = zero16
            return carry

        lax.fori_loop(0, RPS, zbody, 0)
        pltpu.sync_copy(db, acc_sh.at[pl.ds(sid * RPS, RPS)])
        plsc.subcore_barrier()

        def body(j, carry):
            pltpu.sync_copy(ones_v, acc_sh.at[dst_v.at[j]], add=True)
            return carry

        lax.fori_loop(0, NB, body, 0)
        plsc.subcore_barrier()
        pltpu.sync_copy(acc_sh.at[pl.ds(sid * RPS, RPS)], db)

        def rbody(r, carry):
            db[r] = _rsqrt16(db[r] + 1.0)
            return carry

        lax.fori_loop(0, RPS, rbody, 0)

        @pl.when(cid == 0)
        def _():
            pltpu.sync_copy(db, out_hbm.at[sid])

    return dinv


_R = 2000


def _proj_body(x_ref, w1_ref, h0_ref):
    v = lax.dot_general(x_ref[...], w1_ref[...], (((1,), (1,)), ((), ())),
                        preferred_element_type=jnp.float32)
    h0_ref[0] = v[:, :D_HALF]
    h0_ref[1] = v[:, D_HALF:]


def _out_body(g0_ref, g1_ref, w2_ref, b2_ref, w3_ref, b3_ref, m_ref, s_ref):
    g = jnp.concatenate([g0_ref[0], g1_ref[0]], axis=-1)
    dims = (((1,), (1,)), ((), ()))
    m_ref[...] = lax.dot_general(g, w2_ref[...], dims,
                                 preferred_element_type=jnp.float32) + b2_ref[...]
    s_ref[...] = lax.dot_general(g, w3_ref[...], dims,
                                 preferred_element_type=jnp.float32) + b3_ref[...]


def _row_spec(d):
    return pl.BlockSpec((_R, d), lambda i: (i, 0))


def _part_spec(c, d):
    return pl.BlockSpec((1, _R, d), lambda i, _c=c: (_c, i, 0))


def _full_spec(shape):
    return pl.BlockSpec(shape, lambda i: (0,) * len(shape))


def _proj(x, W1):
    return pl.pallas_call(
        _proj_body,
        grid=(N // _R,),
        in_specs=[_row_spec(D_IN), _full_spec(W1.shape)],
        out_specs=pl.BlockSpec((NC, _R, D_HALF), lambda i: (0, i, 0)),
        out_shape=jax.ShapeDtypeStruct((NC, N_ACC, D_HALF), jnp.float32),
    )(x, W1)


def _out_stage(g, W2, b2, W3, b3):
    return pl.pallas_call(
        _out_body,
        grid=(N // _R,),
        in_specs=[_part_spec(0, D_HALF), _part_spec(1, D_HALF),
                  _full_spec(W2.shape), _full_spec(b2.shape),
                  _full_spec(W3.shape), _full_spec(b3.shape)],
        out_specs=[_row_spec(D_IN), _row_spec(D_IN)],
        out_shape=[jax.ShapeDtypeStruct((N, D_IN), jnp.float32),
                   jax.ShapeDtypeStruct((N, D_IN), jnp.float32)],
    )(g, g, W2, b2, W3, b3)


def kernel(x, edge_index, W1, b1, W2, b2, W3, b3):
    src_agg = edge_index[0].astype(jnp.int32).reshape(NS, NB, KA)
    dst_agg = edge_index[1].astype(jnp.int32).reshape(NS, NB, KA)

    ones16 = jnp.ones((KA, 16), jnp.float32)
    b1c = b1.reshape(NC, D_HALF)
    shape5 = (NC, NS, NCHUNK, CH, D_HALF)

    dinv = _make_dinv()(dst_agg, ones16)
    h0 = _proj(x, W1)
    h0r = h0.reshape(shape5)

    p2 = _make_agg(True, "p2")(h0r, src_agg, dst_agg, dinv, b1c)
    g = _make_agg(False, "g")(p2, src_agg, dst_agg, dinv, b1c)

    m, s = _out_stage(g.reshape(NC, N_ACC, D_HALF),
                      W2, b2.reshape(1, D_IN), W3, b3.reshape(1, D_IN))
    return (m, s)


# speedup vs baseline: 1.0635x; 1.0425x over previous
import functools

import jax
import jax.numpy as jnp
from jax import lax
from jax.experimental import pallas as pl
from jax.experimental.pallas import tpu as pltpu
from jax.experimental.pallas import tpu_sc as plsc

N = 10000
E = 320000
D_IN = 128
D_HID = 64
D_HALF = D_HID // 2

NC = 2
NS = 16
KA = 160
NB = E // (NS * KA)
N_ACC = 10048
RPS = N_ACC // NS
NCHUNK = 4
CH = RPS // NCHUNK
L = 16


def _sc_mesh():
    return plsc.VectorSubcoreMesh(core_axis_name="c", subcore_axis_name="s",
                                  num_cores=NC, num_subcores=NS)


def _rsqrt16(x):
    i = plsc.bitcast(x, jnp.int32)
    i = jnp.int32(0x5F3759DF) - (i >> 1)
    y = plsc.bitcast(i, jnp.float32)
    y = y * (1.5 - 0.5 * x * y * y)
    y = y * (1.5 - 0.5 * x * y * y)
    return y


@functools.lru_cache(maxsize=None)
def _make_agg(scale_stage, epi):
    NBUF = 4
    PF = 2

    @functools.partial(
        pl.kernel,
        out_type=jax.ShapeDtypeStruct((NC, NS, NCHUNK, CH, D_HALF),
                                      jnp.float32),
        mesh=_sc_mesh(),
        compiler_params=pltpu.CompilerParams(use_tc_tiling_on_sc=False,
                                             needs_layout_passes=False),
        scratch_types=(
            [pltpu.VMEM((NB, KA), jnp.int32)] * 2
            + [pltpu.VMEM((KA, D_HALF), jnp.float32)] * NBUF
            + [pltpu.VMEM((RPS, 16), jnp.float32)]
            + [pltpu.VMEM((CH, D_HALF), jnp.float32)] * 2
            + [pltpu.VMEM((D_HALF,), jnp.float32)]
            + [pltpu.VMEM_SHARED((N_ACC, D_HALF), jnp.float32)] * 2
            + [pltpu.SemaphoreType.DMA] * (2 * NBUF)
        ),
    )
    def agg(p_hbm, ei_hbm, dinv_hbm, b_hbm, out_hbm, src_v, dst_v, *rest):
        rows = rest[0:NBUF]
        dv_v = rest[NBUF]
        cb0, cb1 = rest[NBUF + 1], rest[NBUF + 2]
        b_v = rest[NBUF + 3]
        acc_sh = rest[NBUF + 4]
        tab = rest[NBUF + 5]
        gsem = rest[NBUF + 6:2 * NBUF + 6]
        ssem = rest[2 * NBUF + 6:3 * NBUF + 6]
        cid = lax.axis_index("c")
        sid = lax.axis_index("s")
        base = sid * RPS
        pltpu.sync_copy(ei_hbm.at[0, sid], src_v)
        pltpu.sync_copy(ei_hbm.at[1, sid], dst_v)
        pltpu.sync_copy(dinv_hbm.at[sid], dv_v)
        pltpu.sync_copy(b_hbm.at[cid], b_v)

        zero16 = jnp.zeros((L,), jnp.float32)

        def zbody(r, carry):
            cb1[r, pl.ds(0, L)] = zero16
            cb1[r, pl.ds(L, L)] = zero16
            return carry

        lax.fori_loop(0, CH, zbody, 0)
        for ck in range(NCHUNK):
            pltpu.sync_copy(cb1, acc_sh.at[pl.ds(base + ck * CH, CH)])

        for ck in range(NCHUNK):
            off = base + ck * CH
            if scale_stage:
                pltpu.sync_copy(p_hbm.at[cid, sid, ck], cb0)

                def sbody(r, carry, _ck=ck):
                    dv = dv_v[_ck * CH + r]
                    cb0[r, pl.ds(0, L)] = cb0[r, pl.ds(0, L)] * dv
                    cb0[r, pl.ds(L, L)] = cb0[r, pl.ds(L, L)] * dv
                    return carry

                lax.fori_loop(0, CH, sbody, 0)
                pltpu.sync_copy(cb0, tab.at[pl.ds(off, CH)])
            else:
                pltpu.sync_copy(p_hbm.at[cid, sid, ck], tab.at[pl.ds(off, CH)])
        plsc.subcore_barrier()

        def gather(j, b):
            pltpu.async_copy(tab.at[src_v.at[j]], rows[b], gsem[b])

        def gwait(j, b):
            pltpu.make_async_copy(tab.at[src_v.at[j]], rows[b], gsem[b]).wait()

        def scat(j, b):
            pltpu.async_copy(rows[b], acc_sh.at[dst_v.at[j]], ssem[b], add=True)

        def swait(j, b):
            pltpu.make_async_copy(rows[b], acc_sh.at[dst_v.at[j]],
                                  ssem[b]).wait()

        for j in range(PF):
            gather(j, j % NBUF)
        for j in range(NBUF):
            if j + PF < NB:
                bp = (j + PF) % NBUF
                if j + PF >= NBUF:
                    swait(j + PF - NBUF, bp)
                gather(j + PF, bp)
            gwait(j, j % NBUF)
            scat(j, j % NBUF)

        def body(t, carry):
            for b in range(NBUF):
                j = NBUF * t + b
                bp = (b + PF) % NBUF
                swait(j + PF - NBUF, bp)
                gather(j + PF, bp)
                gwait(j, b)
                scat(j, b)
            return carry

        n_main = (NB - PF) // NBUF
        lax.fori_loop(1, n_main, body, 0)
        for j in range(NBUF * n_main, NB):
            b = j % NBUF
            if j + PF < NB:
                bp = (j + PF) % NBUF
                swait(j + PF - NBUF, bp)
                gather(j + PF, bp)
            gwait(j, b)
            scat(j, b)
        for j in range(NB - NBUF, NB):
            swait(j, j % NBUF)
        plsc.subcore_barrier()

        for ck in range(NCHUNK):
            off = base + ck * CH
            pltpu.sync_copy(acc_sh.at[pl.ds(off, CH)], cb0)
            pltpu.sync_copy(tab.at[pl.ds(off, CH)], cb1)

            def ebody(r, carry, _ck=ck):
                dv = dv_v[_ck * CH + r]
                for half in range(2):
                    sl = pl.ds(half * L, L)
                    u = cb0[r, sl] + cb1[r, sl]
                    if epi == "p2":
                        v = (u * dv + b_v[sl]) * dv
                    else:
                        v = u * dv
                    cb0[r, sl] = v
                return carry

            lax.fori_loop(0, CH, ebody, 0)
            pltpu.sync_copy(cb0, out_hbm.at[cid, sid, ck])

    return agg


@functools.lru_cache(maxsize=None)
def _make_dinv():
    @functools.partial(
        pl.kernel,
        out_type=jax.ShapeDtypeStruct((NS, RPS, 16), jnp.float32),
        mesh=_sc_mesh(),
        compiler_params=pltpu.CompilerParams(use_tc_tiling_on_sc=False,
                                             needs_layout_passes=False),
        scratch_types=[
            pltpu.VMEM((NB, KA), jnp.int32),
            pltpu.VMEM((KA, 16), jnp.float32),
            pltpu.VMEM((RPS, 16), jnp.float32),
            pltpu.VMEM_SHARED((N_ACC, 16), jnp.float32),
        ],
    )
    def dinv(ei_hbm, ones_hbm, out_hbm, dst_v, ones_v, db, acc_sh):
        cid = lax.axis_index("c")
        sid = lax.axis_index("s")
        pltpu.sync_copy(ei_hbm.at[1, sid], dst_v)
        pltpu.sync_copy(ones_hbm, ones_v)

        zero16 = jnp.zeros((16,), jnp.float32)

        def zbody(r, carry):
            db[r] = zero16
            return carry

        lax.fori_loop(0, RPS, zbody, 0)
        pltpu.sync_copy(db, acc_sh.at[pl.ds(sid * RPS, RPS)])
        plsc.subcore_barrier()

        def body(j, carry):
            pltpu.sync_copy(ones_v, acc_sh.at[dst_v.at[j]], add=True)
            return carry

        lax.fori_loop(0, NB, body, 0)
        plsc.subcore_barrier()
        pltpu.sync_copy(acc_sh.at[pl.ds(sid * RPS, RPS)], db)

        def rbody(r, carry):
            db[r] = _rsqrt16(db[r] + 1.0)
            return carry

        lax.fori_loop(0, RPS, rbody, 0)

        @pl.when(cid == 0)
        def _():
            pltpu.sync_copy(db, out_hbm.at[sid])

    return dinv


_R = 2000


def _proj_body(x_ref, w1_ref, h0_ref):
    v = lax.dot_general(x_ref[...], w1_ref[...], (((1,), (1,)), ((), ())),
                        preferred_element_type=jnp.float32)
    h0_ref[0] = v[:, :D_HALF]
    h0_ref[1] = v[:, D_HALF:]


def _out_body(g0_ref, g1_ref, w2_ref, b2_ref, w3_ref, b3_ref, m_ref, s_ref):
    g = jnp.concatenate([g0_ref[0], g1_ref[0]], axis=-1)
    dims = (((1,), (1,)), ((), ()))
    m_ref[...] = lax.dot_general(g, w2_ref[...], dims,
                                 preferred_element_type=jnp.float32) + b2_ref[...]
    s_ref[...] = lax.dot_general(g, w3_ref[...], dims,
                                 preferred_element_type=jnp.float32) + b3_ref[...]


def _row_spec(d):
    return pl.BlockSpec((_R, d), lambda i: (i, 0))


def _part_spec(c, d):
    return pl.BlockSpec((1, _R, d), lambda i, _c=c: (_c, i, 0))


def _full_spec(shape):
    return pl.BlockSpec(shape, lambda i: (0,) * len(shape))


def _proj(x, W1):
    return pl.pallas_call(
        _proj_body,
        grid=(N // _R,),
        in_specs=[_row_spec(D_IN), _full_spec(W1.shape)],
        out_specs=pl.BlockSpec((NC, _R, D_HALF), lambda i: (0, i, 0)),
        out_shape=jax.ShapeDtypeStruct((NC, N_ACC, D_HALF), jnp.float32),
    )(x, W1)


def _out_stage(g, W2, b2, W3, b3):
    return pl.pallas_call(
        _out_body,
        grid=(N // _R,),
        in_specs=[_part_spec(0, D_HALF), _part_spec(1, D_HALF),
                  _full_spec(W2.shape), _full_spec(b2.shape),
                  _full_spec(W3.shape), _full_spec(b3.shape)],
        out_specs=[_row_spec(D_IN), _row_spec(D_IN)],
        out_shape=[jax.ShapeDtypeStruct((N, D_IN), jnp.float32),
                   jax.ShapeDtypeStruct((N, D_IN), jnp.float32)],
    )(g, g, W2, b2, W3, b3)


def kernel(x, edge_index, W1, b1, W2, b2, W3, b3):
    ei = edge_index.astype(jnp.int32).reshape(2, NS, NB, KA)

    ones16 = jnp.ones((KA, 16), jnp.float32)
    b1c = b1.reshape(NC, D_HALF)
    shape5 = (NC, NS, NCHUNK, CH, D_HALF)

    dinv = _make_dinv()(ei, ones16)
    h0 = _proj(x, W1)
    h0r = h0.reshape(shape5)

    p2 = _make_agg(True, "p2")(h0r, ei, dinv, b1c)
    g = _make_agg(False, "g")(p2, ei, dinv, b1c)

    m, s = _out_stage(g.reshape(NC, N_ACC, D_HALF),
                      W2, b2.reshape(1, D_IN), W3, b3.reshape(1, D_IN))
    return (m, s)
